# baseline (speedup 1.0000x reference)
import functools

import jax
import jax.numpy as jnp
from jax import lax
from jax.experimental import pallas as pl
from jax.experimental.pallas import tpu as pltpu
from jax.experimental.pallas import tpu_sc as plsc

_NUM_CORES = 2
_NUM_SUBCORES = 16
_NUM_WORKERS = _NUM_CORES * _NUM_SUBCORES
_CHUNK = 1024
_G = 250112
_NB = _G // 128


def _tc_relayout(emb_t):
    dim, V = emb_t.shape

    def body(i0, i1, i2, i3, out_ref):
        for p, r in enumerate((i0, i1, i2, i3)):
            out_ref[:, p * dim:(p + 1) * dim] = r[...].T

    return pl.pallas_call(
        body,
        grid=(_NB,),
        in_specs=[
            pl.BlockSpec((dim, 128), (lambda i, p=p: (0, p * _NB + i)))
            for p in range(4)
        ],
        out_specs=pl.BlockSpec((128, 128), lambda i: (i, 0)),
        out_shape=jax.ShapeDtypeStruct((_G, 128), jnp.float32),
    )(emb_t, emb_t, emb_t, emb_t)


@jax.jit
def _embed(embeddings, x):
    V, dim = embeddings.shape
    L, _, B = x.shape
    n = L * B
    per_worker = n // _NUM_WORKERS
    nchunks = per_worker // _CHUNK

    emb_t = jnp.transpose(embeddings)
    scratch = _tc_relayout(emb_t)
    table_lin = jnp.reshape(scratch, (_G * 4, dim))
    x2 = (x % _G) * 4 + x // _G

    mesh = plsc.VectorSubcoreMesh(core_axis_name="c", subcore_axis_name="s")

    @functools.partial(
        pl.kernel,
        mesh=mesh,
        out_type=jax.ShapeDtypeStruct((L, B, dim), jnp.float32),
        compiler_params=pltpu.CompilerParams(use_tc_tiling_on_sc=False),
        scratch_types=[
            pltpu.VMEM((_CHUNK,), jnp.int32),
            pltpu.VMEM((_CHUNK,), jnp.int32),
            pltpu.VMEM((_CHUNK, dim), jnp.float32),
            pltpu.VMEM((_CHUNK, dim), jnp.float32),
            pltpu.SemaphoreType.DMA,
            pltpu.SemaphoreType.DMA,
            pltpu.SemaphoreType.DMA,
            pltpu.SemaphoreType.DMA,
        ],
    )
    def k(table_hbm, x_hbm, out_hbm, i0, i1, r0, r1, g0, g1, w0, w1):
        wid = lax.axis_index("s") * _NUM_CORES + lax.axis_index("c")
        base = wid * per_worker
        bufs = ((i0, r0, g0, w0), (i1, r1, g1, w1))

        def fire(c):
            idx_v, rows_v, gsem, _ = bufs[c % 2]
            off = base + c * _CHUNK
            pltpu.sync_copy(x_hbm.at[off // B, 0, pl.ds(off % B, _CHUNK)], idx_v)
            pltpu.async_copy(table_hbm.at[idx_v], rows_v, gsem)

        def drain_gather_start_write(c):
            idx_v, rows_v, gsem, wsem = bufs[c % 2]
            off = base + c * _CHUNK
            pltpu.make_async_copy(table_hbm.at[idx_v], rows_v, gsem).wait()
            pltpu.async_copy(
                rows_v, out_hbm.at[off // B, pl.ds(off % B, _CHUNK), :], wsem
            )

        def drain_write(c):
            _, rows_v, _, wsem = bufs[c % 2]
            off = base + c * _CHUNK
            pltpu.make_async_copy(
                rows_v, out_hbm.at[off // B, pl.ds(off % B, _CHUNK), :], wsem
            ).wait()

        for c in range(nchunks):
            if c >= 2:
                drain_write(c - 2)
            fire(c)
            if c >= 1:
                drain_gather_start_write(c - 1)
        drain_gather_start_write(nchunks - 1)
        drain_write(nchunks - 2)
        drain_write(nchunks - 1)

    return k(table_lin, x2)


def kernel(x, embeddings):
    return _embed(embeddings, x)


# speedup vs baseline: 1.7362x; 1.7362x over previous
import functools

import jax
import jax.numpy as jnp
from jax import lax
from jax.experimental import pallas as pl
from jax.experimental.pallas import tpu as pltpu
from jax.experimental.pallas import tpu_sc as plsc

_NUM_CORES = 2
_NUM_SUBCORES = 16
_NUM_WORKERS = _NUM_CORES * _NUM_SUBCORES


@functools.partial(jax.jit, static_argnames=("chunk",))
def _sc_gather(embeddings, idx, chunk):
    n = idx.shape[0]
    dim = embeddings.shape[1]
    per_worker = n // _NUM_WORKERS
    nchunks = per_worker // chunk
    mesh = plsc.VectorSubcoreMesh(core_axis_name="c", subcore_axis_name="s")

    @functools.partial(
        pl.kernel,
        mesh=mesh,
        out_type=jax.ShapeDtypeStruct((n, dim), jnp.float32),
        compiler_params=pltpu.CompilerParams(use_tc_tiling_on_sc=False),
        scratch_types=[
            pltpu.VMEM((chunk,), jnp.int32),
            pltpu.VMEM((chunk, dim), jnp.float32),
            pltpu.SemaphoreType.DMA,
        ],
    )
    def k(table_hbm, idx_hbm, out_hbm, idx_v, rows_v, sem):
        wid = lax.axis_index("s") * _NUM_CORES + lax.axis_index("c")
        base = wid * per_worker

        @pl.loop(0, nchunks)
        def _(c):
            off = base + c * chunk
            pltpu.sync_copy(idx_hbm.at[pl.ds(off, chunk)], idx_v)
            pltpu.async_copy(table_hbm.at[idx_v], rows_v, sem).wait()
            pltpu.sync_copy(rows_v, out_hbm.at[pl.ds(off, chunk)])

    return k(embeddings, idx)


def kernel(x, embeddings):
    L, _, B = x.shape
    n = L * B
    idx = x.reshape(n)
    out = _sc_gather(embeddings, idx, 3200)
    return out.reshape(L, B, embeddings.shape[1])
